# baseline (speedup 1.0000x reference)
import functools

import jax
import jax.numpy as jnp
from jax.experimental import pallas as pl
from jax.experimental.pallas import tpu as pltpu

N = 16384
H = 2048
D = 64


def _fused_kernel(nh, th, tar_ref, embs_ref, src_ref, o_ref, acc_ref):
    k = pl.program_id(0)

    @pl.when(k < nh)
    def _phase1():
        acc_ref[pl.ds(k * th, th), :] = jnp.dot(
            tar_ref[...], embs_ref[...], preferred_element_type=jnp.float32)

    @pl.when(k >= nh)
    def _phase2():
        o_ref[...] = jnp.maximum(
            jnp.dot(src_ref[...], acc_ref[...],
                    preferred_element_type=jnp.float32),
            0.0)


@functools.partial(jax.jit, static_argnames=("th", "tm"))
def _run(pois_embs, HG_poi_src, HG_poi_tar, th=128, tm=1024):
    nh = H // th
    nm = N // tm
    return pl.pallas_call(
        functools.partial(_fused_kernel, nh, th),
        grid=(nh + nm,),
        in_specs=[
            pl.BlockSpec((th, N), lambda k: (jnp.minimum(k, nh - 1), 0)),
            pl.BlockSpec((N, D), lambda k: (0, 0)),
            pl.BlockSpec((tm, H), lambda k: (jnp.maximum(k - nh, 0), 0)),
        ],
        out_specs=pl.BlockSpec((tm, D), lambda k: (jnp.maximum(k - nh, 0), 0)),
        out_shape=jax.ShapeDtypeStruct((N, D), jnp.float32),
        scratch_shapes=[pltpu.VMEM((H, D), jnp.float32)],
        compiler_params=pltpu.CompilerParams(
            dimension_semantics=("arbitrary",),
            vmem_limit_bytes=63 * 1024 * 1024),
    )(HG_poi_tar, pois_embs, HG_poi_src)


def kernel(pois_embs, HG_poi_src, HG_poi_tar):
    return _run(pois_embs, HG_poi_src, HG_poi_tar)
